# Initial kernel scaffold; baseline (speedup 1.0000x reference)
import jax
import jax.numpy as jnp
from jax.experimental import pallas as pl


def kernel(X, edge_index, weights):
    raise NotImplementedError("write your pallas kernel here")


# speedup vs baseline: 6.0253x; 6.0253x over previous
import functools

import jax
import jax.numpy as jnp
from jax import lax
from jax.experimental import pallas as pl
from jax.experimental.pallas import tpu as pltpu
from jax.experimental.pallas import tpu_sc as plsc

N = 10000
E = 160000
D = 256
DH = 128
NC = 2
NS = 16
CHUNK = 128
NP = 10240
EP = 163840
ROWS_PER_TILE = NP // NS
EDGES_PER_TILE_K1 = EP // (NC * NS)
EDGES_PER_TILE_K4 = EP // NS
NCHUNK_K1 = EDGES_PER_TILE_K1 // CHUNK
NCHUNK_K4 = EDGES_PER_TILE_K4 // CHUNK

_mesh = plsc.VectorSubcoreMesh(core_axis_name="c", subcore_axis_name="s")


@functools.partial(
    pl.kernel,
    out_type=jax.ShapeDtypeStruct((NC, NP, DH), jnp.float32),
    mesh=_mesh,
    scratch_types=[
        pltpu.VMEM((CHUNK,), jnp.int32),
        pltpu.VMEM((CHUNK, DH), jnp.float32),
        pltpu.VMEM_SHARED((NP, DH), jnp.float32),
    ],
)
def _deg_kernel(dst_hbm, ones_hbm, zeros_hbm, acc_out_hbm, idx_v, ones_v,
                acc_sh):
    c = lax.axis_index("c")
    s = lax.axis_index("s")
    row0 = s * ROWS_PER_TILE
    pltpu.sync_copy(zeros_hbm.at[pl.ds(row0, ROWS_PER_TILE)],
                    acc_sh.at[pl.ds(row0, ROWS_PER_TILE)])
    pltpu.sync_copy(ones_hbm, ones_v)
    plsc.subcore_barrier()

    base = (c * NS + s) * EDGES_PER_TILE_K1

    @pl.loop(0, NCHUNK_K1)
    def _(j):
        pltpu.sync_copy(dst_hbm.at[pl.ds(base + j * CHUNK, CHUNK)], idx_v)
        pltpu.sync_copy(ones_v, acc_sh.at[idx_v], add=True)

    plsc.subcore_barrier()
    pltpu.sync_copy(acc_sh.at[pl.ds(row0, ROWS_PER_TILE)],
                    acc_out_hbm.at[c].at[pl.ds(row0, ROWS_PER_TILE)])


@functools.partial(
    pl.kernel,
    out_type=jax.ShapeDtypeStruct((NC, NP, DH), jnp.float32),
    mesh=_mesh,
    scratch_types=[
        pltpu.VMEM((CHUNK,), jnp.int32),
        pltpu.VMEM((CHUNK,), jnp.int32),
        pltpu.VMEM((CHUNK, DH), jnp.float32),
        pltpu.VMEM_SHARED((NP, DH), jnp.float32),
    ],
)
def _agg_kernel(hs_hbm, src_hbm, dst_hbm, zeros_hbm, r_out_hbm, sidx_v,
                didx_v, rows_v, acc_sh):
    c = lax.axis_index("c")
    s = lax.axis_index("s")
    row0 = s * ROWS_PER_TILE
    pltpu.sync_copy(zeros_hbm.at[pl.ds(row0, ROWS_PER_TILE)],
                    acc_sh.at[pl.ds(row0, ROWS_PER_TILE)])
    plsc.subcore_barrier()

    base = s * EDGES_PER_TILE_K4
    table = hs_hbm.at[c]

    @pl.loop(0, NCHUNK_K4)
    def _(j):
        off = base + j * CHUNK
        pltpu.sync_copy(src_hbm.at[pl.ds(off, CHUNK)], sidx_v)
        pltpu.sync_copy(dst_hbm.at[pl.ds(off, CHUNK)], didx_v)
        pltpu.sync_copy(table.at[sidx_v], rows_v)
        pltpu.sync_copy(rows_v, acc_sh.at[didx_v], add=True)

    plsc.subcore_barrier()
    pltpu.sync_copy(acc_sh.at[pl.ds(row0, ROWS_PER_TILE)],
                    r_out_hbm.at[c].at[pl.ds(row0, ROWS_PER_TILE)])


_RB = 1000


def _mm_body(x_ref, w_ref, o_ref):
    o_ref[...] = jnp.dot(x_ref[...], w_ref[...],
                         preferred_element_type=jnp.float32)


def _matmul(X, W):
    return pl.pallas_call(
        _mm_body,
        grid=(N // _RB,),
        in_specs=[
            pl.BlockSpec((_RB, D), lambda i: (i, 0)),
            pl.BlockSpec((D, D), lambda i: (0, 0)),
        ],
        out_specs=pl.BlockSpec((_RB, D), lambda i: (i, 0)),
        out_shape=jax.ShapeDtypeStruct((N, D), jnp.float32),
    )(X, W)


def _norm_from_acc(acc_ref):
    deg = acc_ref[0, :, 0] + acc_ref[1, :, 0]
    return lax.rsqrt(jnp.maximum(deg, 1.0))


def _scale_body(h_ref, acc_ref, o_ref):
    norm = _norm_from_acc(acc_ref)[:, None]
    h = h_ref[...]
    o_ref[0] = h[:, :DH] * norm
    o_ref[1] = h[:, DH:] * norm


def _scale(H, acc):
    return pl.pallas_call(
        _scale_body,
        grid=(N // _RB,),
        in_specs=[
            pl.BlockSpec((_RB, D), lambda i: (i, 0)),
            pl.BlockSpec((NC, _RB, DH), lambda i: (0, i, 0)),
        ],
        out_specs=pl.BlockSpec((NC, _RB, DH), lambda i: (0, i, 0)),
        out_shape=jax.ShapeDtypeStruct((NC, N, DH), jnp.float32),
    )(H, acc)


def _final_body(r_ref, acc_ref, o_ref):
    norm = _norm_from_acc(acc_ref)[:, None]
    o_ref[:, :DH] = r_ref[0] * norm
    o_ref[:, DH:] = r_ref[1] * norm


def _final(R, acc):
    return pl.pallas_call(
        _final_body,
        grid=(N // _RB,),
        in_specs=[
            pl.BlockSpec((NC, _RB, DH), lambda i: (0, i, 0)),
            pl.BlockSpec((NC, _RB, DH), lambda i: (0, i, 0)),
        ],
        out_specs=pl.BlockSpec((_RB, D), lambda i: (i, 0)),
        out_shape=jax.ShapeDtypeStruct((N, D), jnp.float32),
    )(R, acc)


def kernel(X, edge_index, weights):
    src = edge_index[0].astype(jnp.int32)
    dst = edge_index[1].astype(jnp.int32)
    pad = EP - E
    srcp = jnp.concatenate([src, jnp.zeros((pad,), jnp.int32)])
    dstp = jnp.concatenate([dst, jnp.full((pad,), N, jnp.int32)])

    ones128 = jnp.ones((CHUNK, DH), jnp.float32)
    zeros128 = jnp.zeros((NP, DH), jnp.float32)

    acc = _deg_kernel(dstp, ones128, zeros128)
    H = _matmul(X, weights)
    hs = _scale(H, acc[:, :N, :])
    R = _agg_kernel(hs, srcp, dstp, zeros128)
    out = _final(R[:, :N, :], acc[:, :N, :])
    return out


# speedup vs baseline: 7.4930x; 1.2436x over previous
import functools

import jax
import jax.numpy as jnp
from jax import lax
from jax.experimental import pallas as pl
from jax.experimental.pallas import tpu as pltpu
from jax.experimental.pallas import tpu_sc as plsc

N = 10000
E = 160000
D = 256
DH = 128
NC = 2
NS = 16
CHUNK = 128
NP = 10240
EP = 163840
ROWS_PER_TILE = NP // NS
EDGES_PER_TILE_K1 = EP // (NC * NS)
EDGES_PER_TILE_K4 = EP // NS
NCHUNK_K1 = EDGES_PER_TILE_K1 // CHUNK
NCHUNK_K4 = EDGES_PER_TILE_K4 // CHUNK

_mesh = plsc.VectorSubcoreMesh(core_axis_name="c", subcore_axis_name="s")


@functools.partial(
    pl.kernel,
    out_type=jax.ShapeDtypeStruct((NC, NP, DH), jnp.float32),
    mesh=_mesh,
    scratch_types=[
        pltpu.VMEM((CHUNK,), jnp.int32),
        pltpu.VMEM((CHUNK, DH), jnp.float32),
        pltpu.VMEM_SHARED((NP, DH), jnp.float32),
    ],
)
def _deg_kernel(dst_hbm, ones_hbm, zeros_hbm, acc_out_hbm, idx_v, ones_v,
                acc_sh):
    c = lax.axis_index("c")
    s = lax.axis_index("s")
    row0 = s * ROWS_PER_TILE
    pltpu.sync_copy(zeros_hbm.at[pl.ds(row0, ROWS_PER_TILE)],
                    acc_sh.at[pl.ds(row0, ROWS_PER_TILE)])
    pltpu.sync_copy(ones_hbm, ones_v)
    plsc.subcore_barrier()

    base = (c * NS + s) * EDGES_PER_TILE_K1

    @pl.loop(0, NCHUNK_K1)
    def _(j):
        pltpu.sync_copy(dst_hbm.at[pl.ds(base + j * CHUNK, CHUNK)], idx_v)
        pltpu.sync_copy(ones_v, acc_sh.at[idx_v], add=True)

    plsc.subcore_barrier()
    pltpu.sync_copy(acc_sh.at[pl.ds(row0, ROWS_PER_TILE)],
                    acc_out_hbm.at[c].at[pl.ds(row0, ROWS_PER_TILE)])


NBUF = 2


@functools.partial(
    pl.kernel,
    out_type=jax.ShapeDtypeStruct((NC, NP, DH), jnp.float32),
    mesh=_mesh,
    scratch_types=[
        pltpu.VMEM((NBUF, CHUNK), jnp.int32),
        pltpu.VMEM((NCHUNK_K4, CHUNK), jnp.int32),
        pltpu.VMEM((NBUF, CHUNK, DH), jnp.float32),
        pltpu.VMEM_SHARED((NP, DH), jnp.float32),
    ] + [pltpu.SemaphoreType.DMA] * (2 * NBUF),
)
def _agg_kernel(hs_hbm, src_hbm, dst_hbm, zeros_hbm, r_out_hbm, sidx_v,
                didx_v, bufs, acc_sh, *sems):
    gsems = sems[:NBUF]
    ssems = sems[NBUF:]
    c = lax.axis_index("c")
    s = lax.axis_index("s")
    row0 = s * ROWS_PER_TILE
    cbase = s * NCHUNK_K4
    pltpu.sync_copy(dst_hbm.at[pl.ds(cbase, NCHUNK_K4)], didx_v)
    pltpu.sync_copy(src_hbm.at[pl.ds(cbase, NBUF)], sidx_v)
    pltpu.sync_copy(zeros_hbm.at[pl.ds(row0, ROWS_PER_TILE)],
                    acc_sh.at[pl.ds(row0, ROWS_PER_TILE)])
    plsc.subcore_barrier()

    table = hs_hbm.at[c]

    def start_gather(b):
        pltpu.async_copy(table.at[sidx_v.at[b]], bufs.at[b], gsems[b])

    def wait_gather(b):
        pltpu.make_async_copy(table.at[sidx_v.at[b]], bufs.at[b],
                              gsems[b]).wait()

    def start_scatter(j, b):
        pltpu.async_copy(bufs.at[b], acc_sh.at[didx_v.at[j]], ssems[b],
                         add=True)

    def wait_scatter(j, b):
        pltpu.make_async_copy(bufs.at[b], acc_sh.at[didx_v.at[j]],
                              ssems[b]).wait()

    for b in range(NBUF):
        start_gather(b)

    @pl.loop(0, NCHUNK_K4 - NBUF, step=NBUF)
    def _(j0):
        for b in range(NBUF):
            wait_gather(b)
            start_scatter(j0 + b, b)
        pltpu.sync_copy(src_hbm.at[pl.ds(cbase + j0 + NBUF, NBUF)], sidx_v)
        for b in range(NBUF):
            wait_scatter(j0 + b, b)
            start_gather(b)

    for b in range(NBUF):
        j = NCHUNK_K4 - NBUF + b
        wait_gather(b)
        start_scatter(j, b)
    for b in range(NBUF):
        wait_scatter(NCHUNK_K4 - NBUF + b, b)

    plsc.subcore_barrier()
    pltpu.sync_copy(acc_sh.at[pl.ds(row0, ROWS_PER_TILE)],
                    r_out_hbm.at[c].at[pl.ds(row0, ROWS_PER_TILE)])


_RB = 1000


def _mm_body(x_ref, w_ref, o_ref):
    o_ref[...] = jnp.dot(x_ref[...], w_ref[...],
                         preferred_element_type=jnp.float32)


def _matmul(X, W):
    return pl.pallas_call(
        _mm_body,
        grid=(N // _RB,),
        in_specs=[
            pl.BlockSpec((_RB, D), lambda i: (i, 0)),
            pl.BlockSpec((D, D), lambda i: (0, 0)),
        ],
        out_specs=pl.BlockSpec((_RB, D), lambda i: (i, 0)),
        out_shape=jax.ShapeDtypeStruct((N, D), jnp.float32),
    )(X, W)


def _norm_from_acc(acc_ref):
    deg = acc_ref[0, :, 0] + acc_ref[1, :, 0]
    return lax.rsqrt(jnp.maximum(deg, 1.0))


def _scale_body(h_ref, acc_ref, o_ref):
    norm = _norm_from_acc(acc_ref)[:, None]
    h = h_ref[...]
    o_ref[0] = h[:, :DH] * norm
    o_ref[1] = h[:, DH:] * norm


def _scale(H, acc):
    return pl.pallas_call(
        _scale_body,
        grid=(N // _RB,),
        in_specs=[
            pl.BlockSpec((_RB, D), lambda i: (i, 0)),
            pl.BlockSpec((NC, _RB, DH), lambda i: (0, i, 0)),
        ],
        out_specs=pl.BlockSpec((NC, _RB, DH), lambda i: (0, i, 0)),
        out_shape=jax.ShapeDtypeStruct((NC, N, DH), jnp.float32),
    )(H, acc)


def _final_body(r_ref, acc_ref, o_ref):
    norm = _norm_from_acc(acc_ref)[:, None]
    o_ref[:, :DH] = r_ref[0] * norm
    o_ref[:, DH:] = r_ref[1] * norm


def _final(R, acc):
    return pl.pallas_call(
        _final_body,
        grid=(N // _RB,),
        in_specs=[
            pl.BlockSpec((NC, _RB, DH), lambda i: (0, i, 0)),
            pl.BlockSpec((NC, _RB, DH), lambda i: (0, i, 0)),
        ],
        out_specs=pl.BlockSpec((_RB, D), lambda i: (i, 0)),
        out_shape=jax.ShapeDtypeStruct((N, D), jnp.float32),
    )(R, acc)


def kernel(X, edge_index, weights):
    src = edge_index[0].astype(jnp.int32)
    dst = edge_index[1].astype(jnp.int32)
    pad = EP - E
    srcp = jnp.concatenate([src, jnp.zeros((pad,), jnp.int32)])
    dstp = jnp.concatenate([dst, jnp.full((pad,), N, jnp.int32)])

    ones128 = jnp.ones((CHUNK, DH), jnp.float32)
    zeros128 = jnp.zeros((NP, DH), jnp.float32)

    srcm = srcp.reshape(EP // CHUNK, CHUNK)
    dstm = dstp.reshape(EP // CHUNK, CHUNK)

    acc = _deg_kernel(dstp, ones128, zeros128)
    H = _matmul(X, weights)
    hs = _scale(H, acc[:, :N, :])
    R = _agg_kernel(hs, srcm, dstm, zeros128)
    out = _final(R[:, :N, :], acc[:, :N, :])
    return out


# speedup vs baseline: 8.2414x; 1.0999x over previous
import dataclasses
import functools

import jax
import jax.numpy as jnp
from jax import lax
from jax.experimental import pallas as pl
from jax.experimental.pallas import tpu as pltpu
from jax.experimental.pallas import tpu_sc as plsc

N = 10000
E = 160000
D = 256
DH = 128
NC = 2
NS = 16
CHUNK = 128
NP = 10240
NH = 80
EP = 163840
ROWS_PER_TILE = NP // NS
EDGES_PER_TILE_K1 = EP // (NC * NS)
EDGES_PER_TILE_K4 = EP // NS
NCHUNK_K4 = EDGES_PER_TILE_K4 // CHUNK

_mesh = plsc.VectorSubcoreMesh(core_axis_name="c", subcore_axis_name="s")

_sc_params = pltpu.CompilerParams()
if "needs_layout_passes" in pltpu.CompilerParams.__dataclass_fields__:
    _sc_params = dataclasses.replace(_sc_params, needs_layout_passes=False)


@functools.partial(
    pl.kernel,
    out_type=jax.ShapeDtypeStruct((NC, NH, DH), jnp.float32),
    mesh=_mesh,
    scratch_types=[
        pltpu.VMEM((EDGES_PER_TILE_K1,), jnp.int32),
        pltpu.VMEM((NH, DH), jnp.float32),
        pltpu.VMEM((NH,), jnp.int32),
        pltpu.VMEM_SHARED((NH, DH), jnp.float32),
    ],
    compiler_params=_sc_params,
)
def _deg_kernel(dst_hbm, zeros_hbm, acc_out_hbm, idx_v, hist_v, iota_v,
                acc_sh):
    c = lax.axis_index("c")
    s = lax.axis_index("s")
    base = (c * NS + s) * EDGES_PER_TILE_K1
    pltpu.sync_copy(dst_hbm.at[pl.ds(base, EDGES_PER_TILE_K1)], idx_v)
    pltpu.sync_copy(zeros_hbm.at[pl.ds(0, NH)], hist_v)

    @pl.when(s < NH // 8)
    def _():
        pltpu.sync_copy(zeros_hbm.at[pl.ds(s * 8, 8)],
                        acc_sh.at[pl.ds(s * 8, 8)])

    @pl.loop(0, NH, step=16)
    def _(k):
        iota_v[pl.ds(k, 16)] = lax.iota(jnp.int32, 16) + k

    ones = jnp.ones((16,), jnp.float32)

    @pl.loop(0, EDGES_PER_TILE_K1, step=16)
    def _(k):
        idx16 = idx_v[pl.ds(k, 16)]
        plsc.addupdate_scatter(hist_v, [idx16 >> 7, idx16 & 127], ones)

    plsc.subcore_barrier()
    pltpu.sync_copy(hist_v, acc_sh.at[iota_v], add=True)
    plsc.subcore_barrier()

    @pl.when(s < NH // 8)
    def _():
        pltpu.sync_copy(acc_sh.at[pl.ds(s * 8, 8)],
                        acc_out_hbm.at[c].at[pl.ds(s * 8, 8)])


NBUF = 2


@functools.partial(
    pl.kernel,
    out_type=jax.ShapeDtypeStruct((NC, NP, DH), jnp.float32),
    mesh=_mesh,
    scratch_types=[
        pltpu.VMEM((NBUF, CHUNK), jnp.int32),
        pltpu.VMEM((NCHUNK_K4, CHUNK), jnp.int32),
        pltpu.VMEM((NBUF, CHUNK, DH), jnp.float32),
        pltpu.VMEM_SHARED((NP, DH), jnp.float32),
    ] + [pltpu.SemaphoreType.DMA] * (2 * NBUF),
)
def _agg_kernel(hs_hbm, src_hbm, dst_hbm, zeros_hbm, r_out_hbm, sidx_v,
                didx_v, bufs, acc_sh, *sems):
    gsems = sems[:NBUF]
    ssems = sems[NBUF:]
    c = lax.axis_index("c")
    s = lax.axis_index("s")
    row0 = s * ROWS_PER_TILE
    cbase = s * NCHUNK_K4
    pltpu.sync_copy(dst_hbm.at[pl.ds(cbase, NCHUNK_K4)], didx_v)
    pltpu.sync_copy(src_hbm.at[pl.ds(cbase, NBUF)], sidx_v)
    pltpu.sync_copy(zeros_hbm.at[pl.ds(row0, ROWS_PER_TILE)],
                    acc_sh.at[pl.ds(row0, ROWS_PER_TILE)])
    plsc.subcore_barrier()

    table = hs_hbm.at[c]

    def start_gather(b):
        pltpu.async_copy(table.at[sidx_v.at[b]], bufs.at[b], gsems[b])

    def wait_gather(b):
        pltpu.make_async_copy(table.at[sidx_v.at[b]], bufs.at[b],
                              gsems[b]).wait()

    def start_scatter(j, b):
        pltpu.async_copy(bufs.at[b], acc_sh.at[didx_v.at[j]], ssems[b],
                         add=True)

    def wait_scatter(j, b):
        pltpu.make_async_copy(bufs.at[b], acc_sh.at[didx_v.at[j]],
                              ssems[b]).wait()

    for b in range(NBUF):
        start_gather(b)

    @pl.loop(0, NCHUNK_K4 - NBUF, step=NBUF)
    def _(j0):
        for b in range(NBUF):
            wait_gather(b)
            start_scatter(j0 + b, b)
        pltpu.sync_copy(src_hbm.at[pl.ds(cbase + j0 + NBUF, NBUF)], sidx_v)
        for b in range(NBUF):
            wait_scatter(j0 + b, b)
            start_gather(b)

    for b in range(NBUF):
        j = NCHUNK_K4 - NBUF + b
        wait_gather(b)
        start_scatter(j, b)
    for b in range(NBUF):
        wait_scatter(NCHUNK_K4 - NBUF + b, b)

    plsc.subcore_barrier()
    pltpu.sync_copy(acc_sh.at[pl.ds(row0, ROWS_PER_TILE)],
                    r_out_hbm.at[c].at[pl.ds(row0, ROWS_PER_TILE)])


_RB = 1000


def _mm_body(x_ref, w_ref, o_ref):
    o_ref[...] = jnp.dot(x_ref[...], w_ref[...],
                         preferred_element_type=jnp.float32)


def _matmul(X, W):
    return pl.pallas_call(
        _mm_body,
        grid=(N // _RB,),
        in_specs=[
            pl.BlockSpec((_RB, D), lambda i: (i, 0)),
            pl.BlockSpec((D, D), lambda i: (0, 0)),
        ],
        out_specs=pl.BlockSpec((_RB, D), lambda i: (i, 0)),
        out_shape=jax.ShapeDtypeStruct((N, D), jnp.float32),
    )(X, W)


def _norm_from_acc(acc_ref):
    deg = acc_ref[0, 0, 0] + acc_ref[1, 0, 0]
    return lax.rsqrt(jnp.maximum(deg, 1.0))


def _scale_body(h_ref, acc_ref, o_ref):
    norm = _norm_from_acc(acc_ref)[:, None]
    h = h_ref[...]
    o_ref[0] = h[:, :DH] * norm
    o_ref[1] = h[:, DH:] * norm


def _scale(H, acc):
    return pl.pallas_call(
        _scale_body,
        grid=(N // _RB,),
        in_specs=[
            pl.BlockSpec((_RB, D), lambda i: (i, 0)),
            pl.BlockSpec((NC, 1, 1, _RB), lambda i: (0, i, 0, 0)),
        ],
        out_specs=pl.BlockSpec((NC, _RB, DH), lambda i: (0, i, 0)),
        out_shape=jax.ShapeDtypeStruct((NC, N, DH), jnp.float32),
    )(H, acc)


def _final_body(r_ref, acc_ref, o_ref):
    norm = _norm_from_acc(acc_ref)[:, None]
    o_ref[:, :DH] = r_ref[0] * norm
    o_ref[:, DH:] = r_ref[1] * norm


def _final(R, acc):
    return pl.pallas_call(
        _final_body,
        grid=(N // _RB,),
        in_specs=[
            pl.BlockSpec((NC, _RB, DH), lambda i: (0, i, 0)),
            pl.BlockSpec((NC, 1, 1, _RB), lambda i: (0, i, 0, 0)),
        ],
        out_specs=pl.BlockSpec((_RB, D), lambda i: (i, 0)),
        out_shape=jax.ShapeDtypeStruct((N, D), jnp.float32),
    )(R, acc)


def kernel(X, edge_index, weights):
    src = edge_index[0].astype(jnp.int32)
    dst = edge_index[1].astype(jnp.int32)
    pad = EP - E
    srcp = jnp.concatenate([src, jnp.zeros((pad,), jnp.int32)])
    dstp = jnp.concatenate([dst, jnp.full((pad,), N, jnp.int32)])

    zeros128 = jnp.zeros((NP, DH), jnp.float32)

    srcm = jnp.concatenate(
        [srcp.reshape(EP // CHUNK, CHUNK),
         jnp.zeros((8, CHUNK), jnp.int32)])
    dstm = jnp.concatenate(
        [dstp.reshape(EP // CHUNK, CHUNK),
         jnp.full((8, CHUNK), N, jnp.int32)])

    acc = _deg_kernel(dstp, zeros128)
    degp = acc.reshape(NC, NH * DH)[:, :N]
    degp = degp.reshape(NC, N // _RB, 1, _RB)
    H = _matmul(X, weights)
    hs = _scale(H, degp)
    R = _agg_kernel(hs, srcm, dstm, zeros128)
    out = _final(R[:, :N, :], degp)
    return out
